# baseline (speedup 1.0000x reference)
import functools

import jax
import jax.numpy as jnp
from jax import lax
from jax.experimental import pallas as pl
from jax.experimental.pallas import tpu as pltpu
from jax.experimental.pallas import tpu_sc as plsc

B = 16384
D = 32
H1 = 128
H2 = 64
V = 1000000

_info = plsc.get_sparse_core_info()
_NC, _NS = _info.num_cores, _info.num_subcores
_NW = _NC * _NS
_BPW = B // _NW

_sc_mesh = plsc.VectorSubcoreMesh(core_axis_name="c", subcore_axis_name="s")


_TK = 8192
_TSTEPS = -(-V // _TK)


def _transpose_body(ut_ref, vt_ref, eye_ref, uo_ref, vo_ref):
    tdot = functools.partial(
        jax.lax.dot_general,
        dimension_numbers=(((0,), (0,)), ((), ())),
        preferred_element_type=jnp.float32,
        precision=jax.lax.Precision.DEFAULT,
    )
    uo_ref[...] = tdot(ut_ref[...], eye_ref[...])
    vo_ref[...] = tdot(vt_ref[...], eye_ref[...])


def _transpose_tc(uT, vT):
    return pl.pallas_call(
        _transpose_body,
        grid=(_TSTEPS,),
        in_specs=[
            pl.BlockSpec((D, _TK), lambda i: (0, i)),
            pl.BlockSpec((D, _TK), lambda i: (0, i)),
            pl.BlockSpec((D, D), lambda i: (0, 0)),
        ],
        out_specs=[
            pl.BlockSpec((_TK, D), lambda i: (i, 0)),
            pl.BlockSpec((_TK, D), lambda i: (i, 0)),
        ],
        out_shape=[
            jax.ShapeDtypeStruct((V, D), jnp.float32),
            jax.ShapeDtypeStruct((V, D), jnp.float32),
        ],
    )(uT, vT, jnp.eye(D, dtype=jnp.float32))


@functools.partial(
    pl.kernel,
    mesh=_sc_mesh,
    compiler_params=pltpu.CompilerParams(use_tc_tiling_on_sc=False),
    out_type=(
        jax.ShapeDtypeStruct((B, D), jnp.float32),
        jax.ShapeDtypeStruct((B, D), jnp.float32),
    ),
    scratch_types=[
        pltpu.VMEM((_BPW,), jnp.int32),
        pltpu.VMEM((_BPW,), jnp.int32),
        pltpu.VMEM((_BPW, D), jnp.float32),
        pltpu.VMEM((_BPW, D), jnp.float32),
        pltpu.SemaphoreType.DMA,
        pltpu.SemaphoreType.DMA,
    ],
)
def _gather_sc(users_hbm, items_hbm, uemb_hbm, vemb_hbm, uout_hbm, vout_hbm,
               uidx_v, vidx_v, urows_v, vrows_v, sem_u, sem_v):
    wid = lax.axis_index("s") * _NC + lax.axis_index("c")
    base = wid * _BPW
    pltpu.sync_copy(users_hbm.at[pl.ds(base, _BPW)], uidx_v)
    pltpu.sync_copy(items_hbm.at[pl.ds(base, _BPW)], vidx_v)
    cu = pltpu.async_copy(uemb_hbm.at[uidx_v], urows_v, sem_u)
    cv = pltpu.async_copy(vemb_hbm.at[vidx_v], vrows_v, sem_v)
    cu.wait()
    cv.wait()
    pltpu.sync_copy(urows_v, uout_hbm.at[pl.ds(base, _BPW)])
    pltpu.sync_copy(vrows_v, vout_hbm.at[pl.ds(base, _BPW)])


_BLK = 2048


def _mlp_body(u_ref, v_ref, w1u_ref, w1v_ref, b1_ref, w2_ref, b2_ref,
              w3t_ref, b3_ref, y_ref):
    dot = functools.partial(
        jax.lax.dot_general,
        dimension_numbers=(((1,), (0,)), ((), ())),
        preferred_element_type=jnp.float32,
        precision=jax.lax.Precision.HIGHEST,
    )
    h = dot(u_ref[...], w1u_ref[...]) + dot(v_ref[...], w1v_ref[...])
    h = jnp.maximum(h + b1_ref[...], 0.0)
    h = jnp.maximum(dot(h, w2_ref[...]) + b2_ref[...], 0.0)
    y_ref[...] = jnp.sum(h * w3t_ref[...], axis=1) + b3_ref[0]


def _mlp_tc(u, v, w1u, w1v, b1, w2, b2, w3t, b3):
    grid = (B // _BLK,)
    full = lambda shape: pl.BlockSpec(shape, lambda i: (0,) * len(shape))
    return pl.pallas_call(
        _mlp_body,
        grid=grid,
        in_specs=[
            pl.BlockSpec((_BLK, D), lambda i: (i, 0)),
            pl.BlockSpec((_BLK, D), lambda i: (i, 0)),
            full((D, H1)),
            full((D, H1)),
            full((1, H1)),
            full((H1, H2)),
            full((1, H2)),
            full((1, H2)),
            pl.BlockSpec(memory_space=pltpu.SMEM),
        ],
        out_specs=pl.BlockSpec((_BLK,), lambda i: (i,)),
        out_shape=jax.ShapeDtypeStruct((B,), jnp.float32),
    )(u, v, w1u, w1v, b1, w2, b2, w3t, b3)


def kernel(users, items, user_emb, item_emb, W1, b1, W2, b2, W3, b3):
    uemb_rm, vemb_rm = _transpose_tc(user_emb.T, item_emb.T)
    u_rows, v_rows = _gather_sc(users, items, uemb_rm, vemb_rm)
    w1u = W1[:D]
    w1v = W1[D:]
    return _mlp_tc(u_rows, v_rows, w1u, w1v, b1.reshape(1, H1),
                   W2, b2.reshape(1, H2), W3.reshape(1, H2), b3)


# speedup vs baseline: 1.0112x; 1.0112x over previous
import functools

import jax
import jax.numpy as jnp
from jax import lax
from jax.experimental import pallas as pl
from jax.experimental.pallas import tpu as pltpu
from jax.experimental.pallas import tpu_sc as plsc

B = 16384
D = 32
H1 = 128
H2 = 64
V = 1000000

_info = plsc.get_sparse_core_info()
_NC, _NS = _info.num_cores, _info.num_subcores
_NW = _NC * _NS
_BPW = B // _NW

_sc_mesh = plsc.VectorSubcoreMesh(core_axis_name="c", subcore_axis_name="s")


_TK = 16384
_TSTEPS = -(-V // _TK)
_NBAND = D // 8


def _transpose_body(u0, u1, u2, u3, v0, v1, v2, v3, eye_ref, uo_ref, vo_ref):
    tdot = functools.partial(
        jax.lax.dot_general,
        dimension_numbers=(((0,), (0,)), ((), ())),
        preferred_element_type=jnp.float32,
        precision=jax.lax.Precision.DEFAULT,
    )
    eye = eye_ref[...]
    ub = jnp.concatenate([u0[...], u1[...], u2[...], u3[...]], axis=0)
    vb = jnp.concatenate([v0[...], v1[...], v2[...], v3[...]], axis=0)
    uo_ref[...] = tdot(ub, eye)
    vo_ref[...] = tdot(vb, eye)


def _transpose_tc(uT, vT):
    def band_spec(s):
        return pl.BlockSpec((8, _TK), lambda i, s=s: (s, i))

    return pl.pallas_call(
        _transpose_body,
        grid=(_TSTEPS,),
        in_specs=(
            [band_spec(s) for s in range(_NBAND)]
            + [band_spec(s) for s in range(_NBAND)]
            + [pl.BlockSpec((D, D), lambda i: (0, 0))]
        ),
        out_specs=[
            pl.BlockSpec((_TK, D), lambda i: (i, 0)),
            pl.BlockSpec((_TK, D), lambda i: (i, 0)),
        ],
        out_shape=[
            jax.ShapeDtypeStruct((V, D), jnp.float32),
            jax.ShapeDtypeStruct((V, D), jnp.float32),
        ],
    )(uT, uT, uT, uT, vT, vT, vT, vT, jnp.eye(D, dtype=jnp.float32))


@functools.partial(
    pl.kernel,
    mesh=_sc_mesh,
    compiler_params=pltpu.CompilerParams(use_tc_tiling_on_sc=False),
    out_type=(
        jax.ShapeDtypeStruct((B, D), jnp.float32),
        jax.ShapeDtypeStruct((B, D), jnp.float32),
    ),
    scratch_types=[
        pltpu.VMEM((_BPW,), jnp.int32),
        pltpu.VMEM((_BPW,), jnp.int32),
        pltpu.VMEM((_BPW, D), jnp.float32),
        pltpu.VMEM((_BPW, D), jnp.float32),
        pltpu.SemaphoreType.DMA,
        pltpu.SemaphoreType.DMA,
    ],
)
def _gather_sc(users_hbm, items_hbm, uemb_hbm, vemb_hbm, uout_hbm, vout_hbm,
               uidx_v, vidx_v, urows_v, vrows_v, sem_u, sem_v):
    wid = lax.axis_index("s") * _NC + lax.axis_index("c")
    base = wid * _BPW
    pltpu.sync_copy(users_hbm.at[pl.ds(base, _BPW)], uidx_v)
    pltpu.sync_copy(items_hbm.at[pl.ds(base, _BPW)], vidx_v)
    cu = pltpu.async_copy(uemb_hbm.at[uidx_v], urows_v, sem_u)
    cv = pltpu.async_copy(vemb_hbm.at[vidx_v], vrows_v, sem_v)
    cu.wait()
    cv.wait()
    pltpu.sync_copy(urows_v, uout_hbm.at[pl.ds(base, _BPW)])
    pltpu.sync_copy(vrows_v, vout_hbm.at[pl.ds(base, _BPW)])


_BLK = 2048


def _mlp_body(u_ref, v_ref, w1u_ref, w1v_ref, b1_ref, w2_ref, b2_ref,
              w3t_ref, b3_ref, y_ref):
    dot = functools.partial(
        jax.lax.dot_general,
        dimension_numbers=(((1,), (0,)), ((), ())),
        preferred_element_type=jnp.float32,
        precision=jax.lax.Precision.HIGHEST,
    )
    h = dot(u_ref[...], w1u_ref[...]) + dot(v_ref[...], w1v_ref[...])
    h = jnp.maximum(h + b1_ref[...], 0.0)
    h = jnp.maximum(dot(h, w2_ref[...]) + b2_ref[...], 0.0)
    y_ref[...] = jnp.sum(h * w3t_ref[...], axis=1) + b3_ref[0]


def _mlp_tc(u, v, w1u, w1v, b1, w2, b2, w3t, b3):
    grid = (B // _BLK,)
    full = lambda shape: pl.BlockSpec(shape, lambda i: (0,) * len(shape))
    return pl.pallas_call(
        _mlp_body,
        grid=grid,
        in_specs=[
            pl.BlockSpec((_BLK, D), lambda i: (i, 0)),
            pl.BlockSpec((_BLK, D), lambda i: (i, 0)),
            full((D, H1)),
            full((D, H1)),
            full((1, H1)),
            full((H1, H2)),
            full((1, H2)),
            full((1, H2)),
            pl.BlockSpec(memory_space=pltpu.SMEM),
        ],
        out_specs=pl.BlockSpec((_BLK,), lambda i: (i,)),
        out_shape=jax.ShapeDtypeStruct((B,), jnp.float32),
    )(u, v, w1u, w1v, b1, w2, b2, w3t, b3)


def kernel(users, items, user_emb, item_emb, W1, b1, W2, b2, W3, b3):
    uemb_rm, vemb_rm = _transpose_tc(user_emb.T, item_emb.T)
    u_rows, v_rows = _gather_sc(users, items, uemb_rm, vemb_rm)
    w1u = W1[:D]
    w1v = W1[D:]
    return _mlp_tc(u_rows, v_rows, w1u, w1v, b1.reshape(1, H1),
                   W2, b2.reshape(1, H2), W3.reshape(1, H2), b3)


# speedup vs baseline: 1.2124x; 1.1990x over previous
import functools

import jax
import jax.numpy as jnp
from jax import lax
from jax.experimental import pallas as pl
from jax.experimental.pallas import tpu as pltpu
from jax.experimental.pallas import tpu_sc as plsc

B = 16384
D = 32
H1 = 128
H2 = 64

_info = plsc.get_sparse_core_info()
_NC, _NS = _info.num_cores, _info.num_subcores
_NW = _NC * _NS
_BPW = B // _NW

_sc_mesh = plsc.VectorSubcoreMesh(core_axis_name="c", subcore_axis_name="s")


@functools.partial(
    pl.kernel,
    mesh=_sc_mesh,
    compiler_params=pltpu.CompilerParams(use_tc_tiling_on_sc=False),
    out_type=(
        jax.ShapeDtypeStruct((B, D), jnp.float32),
        jax.ShapeDtypeStruct((B, D), jnp.float32),
    ),
    scratch_types=[
        pltpu.VMEM((_BPW,), jnp.int32),
        pltpu.VMEM((_BPW,), jnp.int32),
        pltpu.VMEM((_BPW, D), jnp.float32),
        pltpu.VMEM((_BPW, D), jnp.float32),
        pltpu.SemaphoreType.DMA,
        pltpu.SemaphoreType.DMA,
    ],
)
def _gather_sc(users_hbm, items_hbm, uemb_hbm, vemb_hbm, uout_hbm, vout_hbm,
               uidx_v, vidx_v, urows_v, vrows_v, sem_u, sem_v):
    wid = lax.axis_index("s") * _NC + lax.axis_index("c")
    base = wid * _BPW
    pltpu.sync_copy(users_hbm.at[pl.ds(base, _BPW)], uidx_v)
    pltpu.sync_copy(items_hbm.at[pl.ds(base, _BPW)], vidx_v)
    cu = pltpu.async_copy(uemb_hbm.at[uidx_v], urows_v, sem_u)
    cv = pltpu.async_copy(vemb_hbm.at[vidx_v], vrows_v, sem_v)
    cu.wait()
    cv.wait()
    pltpu.sync_copy(urows_v, uout_hbm.at[pl.ds(base, _BPW)])
    pltpu.sync_copy(vrows_v, vout_hbm.at[pl.ds(base, _BPW)])


_BLK = 2048


def _mlp_body(u_ref, v_ref, w1u_ref, w1v_ref, b1_ref, w2_ref, b2_ref,
              w3t_ref, b3_ref, y_ref):
    dot = functools.partial(
        jax.lax.dot_general,
        dimension_numbers=(((1,), (0,)), ((), ())),
        preferred_element_type=jnp.float32,
        precision=jax.lax.Precision.HIGHEST,
    )
    h = dot(u_ref[...], w1u_ref[...]) + dot(v_ref[...], w1v_ref[...])
    h = jnp.maximum(h + b1_ref[...], 0.0)
    h = jnp.maximum(dot(h, w2_ref[...]) + b2_ref[...], 0.0)
    y_ref[...] = jnp.sum(h * w3t_ref[...], axis=1) + b3_ref[0]


def _mlp_tc(u, v, w1u, w1v, b1, w2, b2, w3t, b3):
    grid = (B // _BLK,)
    full = lambda shape: pl.BlockSpec(shape, lambda i: (0,) * len(shape))
    return pl.pallas_call(
        _mlp_body,
        grid=grid,
        in_specs=[
            pl.BlockSpec((_BLK, D), lambda i: (i, 0)),
            pl.BlockSpec((_BLK, D), lambda i: (i, 0)),
            full((D, H1)),
            full((D, H1)),
            full((1, H1)),
            full((H1, H2)),
            full((1, H2)),
            full((1, H2)),
            pl.BlockSpec(memory_space=pltpu.SMEM),
        ],
        out_specs=pl.BlockSpec((_BLK,), lambda i: (i,)),
        out_shape=jax.ShapeDtypeStruct((B,), jnp.float32),
    )(u, v, w1u, w1v, b1, w2, b2, w3t, b3)


def kernel(users, items, user_emb, item_emb, W1, b1, W2, b2, W3, b3):
    u_rows, v_rows = _gather_sc(users, items, user_emb, item_emb)
    w1u = W1[:D]
    w1v = W1[D:]
    return _mlp_tc(u_rows, v_rows, w1u, w1v, b1.reshape(1, H1),
                   W2, b2.reshape(1, H2), W3.reshape(1, H2), b3)
